# baseline (speedup 1.0000x reference)
import functools

import jax
import jax.numpy as jnp
from jax import lax
from jax.experimental import pallas as pl
from jax.experimental.pallas import tpu as pltpu
from jax.experimental.pallas import tpu_sc as plsc

N = 10000
E = 320000
D = 128

NC = 2
NS = 16
K = 128

CH0 = 42
CH1 = 118
CHMAX = CH1
N_CHUNKS = NS * (CH0 + CH1)
E_PAD = N_CHUNKS * K
ROWS_PER_SUB = 632
N_ACC = ROWS_PER_SUB * NS


def _sc_segment_sum(h, src_idx, dst_idx, zeros_init):
    mesh = plsc.VectorSubcoreMesh(core_axis_name="c", subcore_axis_name="s")

    @functools.partial(
        pl.kernel,
        out_type=jax.ShapeDtypeStruct((NC, N_ACC, D), jnp.float32),
        mesh=mesh,
        scratch_types=[
            pltpu.VMEM_SHARED((N_ACC, D), jnp.float32),
            pltpu.VMEM((2, 1, K), jnp.int32),
            pltpu.VMEM((2, 1, K), jnp.int32),
            pltpu.VMEM((2, K, D), jnp.float32),
            pltpu.SemaphoreType.DMA,
            pltpu.SemaphoreType.DMA,
            pltpu.SemaphoreType.DMA,
            pltpu.SemaphoreType.DMA,
            pltpu.SemaphoreType.DMA,
            pltpu.SemaphoreType.DMA,
            pltpu.SemaphoreType.DMA,
            pltpu.SemaphoreType.DMA,
        ],
    )
    def seg_sum(h_hbm, src_hbm, dst_hbm, zero_hbm, out_hbm, acc, src_v, dst_v,
                rows_v, gs0, gs1, ds0, ds1, ss0, ss1, rs0, rs1):
        c = lax.axis_index("c")
        s = lax.axis_index("s")
        wid = c * NS + s
        gsem = (gs0, gs1)
        dsem = (ds0, ds1)
        ssem = (ss0, ss1)
        rsem = (rs0, rs1)

        pltpu.sync_copy(zero_hbm, acc.at[pl.ds(s * ROWS_PER_SUB, ROWS_PER_SUB)])
        plsc.subcore_barrier()

        def prefetch(i, b):
            pltpu.async_copy(src_hbm.at[wid, i], src_v.at[b], rsem[b])
            pltpu.async_copy(dst_hbm.at[wid, i], dst_v.at[b], dsem[b])

        def gather(b):
            pltpu.make_async_copy(src_hbm.at[wid, 0], src_v.at[b],
                                  rsem[b]).wait()
            pltpu.async_copy(h_hbm.at[src_v.at[b, 0]], rows_v.at[b], gsem[b])

        for b in (0, 1):
            prefetch(b, b)
        for b in (0, 1):
            gather(b)

        npairs = jnp.where(c == 0, CH0 // 2, CH1 // 2)

        def pair_body(q, carry):
            for b in (0, 1):
                i = 2 * q + b
                pltpu.make_async_copy(h_hbm.at[src_v.at[b, 0]], rows_v.at[b],
                                      gsem[b]).wait()
                pltpu.make_async_copy(dst_hbm.at[wid, 0], dst_v.at[b],
                                      dsem[b]).wait()
                pltpu.async_copy(rows_v.at[b], acc.at[dst_v.at[b, 0]], ssem[b],
                                 add=True)
                pltpu.make_async_copy(rows_v.at[b], acc.at[dst_v.at[b, 0]],
                                      ssem[b]).wait()
                @pl.when(q < npairs - 1)
                def _():
                    prefetch(i + 2, b)
                    gather(b)
            return carry

        lax.fori_loop(0, npairs, pair_body, 0)
        plsc.subcore_barrier()

        pltpu.sync_copy(acc.at[pl.ds(s * ROWS_PER_SUB, ROWS_PER_SUB)],
                        out_hbm.at[c, pl.ds(s * ROWS_PER_SUB, ROWS_PER_SUB)])

    return seg_sum(h, src_idx, dst_idx, zeros_init)


def _tc_dense(h, parts, W1, b1, W2, b2, gamma, beta):
    def body(h_ref, p_ref, W1_ref, b1_ref, W2_ref, b2_ref, g_ref, bt_ref,
             out_ref):
        hh = h_ref[...]
        x = hh + p_ref[0, :N, :] + p_ref[1, :N, :]
        y = jnp.maximum(
            jnp.dot(x, W1_ref[...], preferred_element_type=jnp.float32)
            + b1_ref[...], 0.0)
        z = (jnp.dot(y, W2_ref[...], preferred_element_type=jnp.float32)
             + b2_ref[...])
        mean = jnp.mean(z, axis=0, keepdims=True)
        zc = z - mean
        var = jnp.mean(zc * zc, axis=0, keepdims=True)
        zn = zc * jax.lax.rsqrt(var + 1e-5) * g_ref[...] + bt_ref[...]
        out_ref[...] = hh + jnp.maximum(zn, 0.0)

    return pl.pallas_call(
        body,
        out_shape=jax.ShapeDtypeStruct((N, D), jnp.float32),
    )(h, parts, W1, b1.reshape(1, D), W2, b2.reshape(1, D),
      gamma.reshape(1, D), beta.reshape(1, D))


def kernel(h, edge_index, W1, b1, W2, b2, gamma, beta):
    pad = E_PAD - E
    src = jnp.concatenate([edge_index[0], jnp.zeros((pad,), jnp.int32)])
    trash = N + jnp.arange(pad, dtype=jnp.int32) % (N_ACC - N)
    dst = jnp.concatenate([edge_index[1], trash])
    nw = NC * NS

    def to_worker_chunks(flat):
        ch = flat.reshape(N_CHUNKS, K)
        c0 = ch[: NS * CH0].reshape(NS, CH0, K)
        c1 = ch[NS * CH0 :].reshape(NS, CH1, K)
        arr = jnp.zeros((nw, CHMAX, K), jnp.int32)
        arr = arr.at[:NS, :CH0].set(c0).at[NS:].set(c1)
        return arr.reshape(nw, CHMAX, 1, K)

    src = to_worker_chunks(src)
    dst = to_worker_chunks(dst)
    zeros_init = jnp.zeros((ROWS_PER_SUB, D), jnp.float32)
    parts = _sc_segment_sum(h, src, dst, zeros_init)
    return _tc_dense(h, parts, W1, b1, W2, b2, gamma, beta)


# speedup vs baseline: 1.0472x; 1.0472x over previous
import functools

import jax
import jax.numpy as jnp
from jax import lax
from jax.experimental import pallas as pl
from jax.experimental.pallas import tpu as pltpu
from jax.experimental.pallas import tpu_sc as plsc

N = 10000
E = 320000
D = 128

NC = 2
NS = 16
K = 128

CH0 = 118
CH1 = 42
CHMAX = CH0
N_CHUNKS = NS * (CH0 + CH1)
E_PAD = N_CHUNKS * K
ROWS_PER_SUB = 632
N_ACC = ROWS_PER_SUB * NS


def _sc_segment_sum(h, src_idx, dst_idx, zeros_init):
    mesh = plsc.VectorSubcoreMesh(core_axis_name="c", subcore_axis_name="s")

    @functools.partial(
        pl.kernel,
        out_type=jax.ShapeDtypeStruct((NC, N_ACC, D), jnp.float32),
        mesh=mesh,
        scratch_types=[
            pltpu.VMEM_SHARED((N_ACC, D), jnp.float32),
            pltpu.VMEM((2, 1, K), jnp.int32),
            pltpu.VMEM((2, 1, K), jnp.int32),
            pltpu.VMEM((2, K, D), jnp.float32),
            pltpu.SemaphoreType.DMA,
            pltpu.SemaphoreType.DMA,
            pltpu.SemaphoreType.DMA,
            pltpu.SemaphoreType.DMA,
            pltpu.SemaphoreType.DMA,
            pltpu.SemaphoreType.DMA,
            pltpu.SemaphoreType.DMA,
            pltpu.SemaphoreType.DMA,
        ],
    )
    def seg_sum(h_hbm, src_hbm, dst_hbm, zero_hbm, out_hbm, acc, src_v, dst_v,
                rows_v, gs0, gs1, ds0, ds1, ss0, ss1, rs0, rs1):
        c = lax.axis_index("c")
        s = lax.axis_index("s")
        wid = c * NS + s
        gsem = (gs0, gs1)
        dsem = (ds0, ds1)
        ssem = (ss0, ss1)
        rsem = (rs0, rs1)

        pltpu.sync_copy(zero_hbm, acc.at[pl.ds(s * ROWS_PER_SUB, ROWS_PER_SUB)])
        plsc.subcore_barrier()

        def prefetch(i, b):
            pltpu.async_copy(src_hbm.at[wid, i], src_v.at[b], rsem[b])
            pltpu.async_copy(dst_hbm.at[wid, i], dst_v.at[b], dsem[b])

        def gather(b):
            pltpu.make_async_copy(src_hbm.at[wid, 0], src_v.at[b],
                                  rsem[b]).wait()
            pltpu.async_copy(h_hbm.at[src_v.at[b, 0]], rows_v.at[b], gsem[b])

        for b in (0, 1):
            prefetch(b, b)
        for b in (0, 1):
            gather(b)

        npairs = jnp.where(c == 0, CH0 // 2, CH1 // 2)

        def pair_body(q, carry):
            for b in (0, 1):
                i = 2 * q + b
                pltpu.make_async_copy(h_hbm.at[src_v.at[b, 0]], rows_v.at[b],
                                      gsem[b]).wait()
                pltpu.make_async_copy(dst_hbm.at[wid, 0], dst_v.at[b],
                                      dsem[b]).wait()
                pltpu.async_copy(rows_v.at[b], acc.at[dst_v.at[b, 0]], ssem[b],
                                 add=True)
                pltpu.make_async_copy(rows_v.at[b], acc.at[dst_v.at[b, 0]],
                                      ssem[b]).wait()
                @pl.when(q < npairs - 1)
                def _():
                    prefetch(i + 2, b)
                    gather(b)
            return carry

        lax.fori_loop(0, npairs, pair_body, 0)
        plsc.subcore_barrier()

        pltpu.sync_copy(acc.at[pl.ds(s * ROWS_PER_SUB, ROWS_PER_SUB)],
                        out_hbm.at[c, pl.ds(s * ROWS_PER_SUB, ROWS_PER_SUB)])

    return seg_sum(h, src_idx, dst_idx, zeros_init)


def _tc_dense(h, parts, W1, b1, W2, b2, gamma, beta):
    def body(h_ref, p_ref, W1_ref, b1_ref, W2_ref, b2_ref, g_ref, bt_ref,
             out_ref):
        hh = h_ref[...]
        x = hh + p_ref[0, :N, :] + p_ref[1, :N, :]
        y = jnp.maximum(
            jnp.dot(x, W1_ref[...], preferred_element_type=jnp.float32)
            + b1_ref[...], 0.0)
        z = (jnp.dot(y, W2_ref[...], preferred_element_type=jnp.float32)
             + b2_ref[...])
        mean = jnp.mean(z, axis=0, keepdims=True)
        zc = z - mean
        var = jnp.mean(zc * zc, axis=0, keepdims=True)
        zn = zc * jax.lax.rsqrt(var + 1e-5) * g_ref[...] + bt_ref[...]
        out_ref[...] = hh + jnp.maximum(zn, 0.0)

    return pl.pallas_call(
        body,
        out_shape=jax.ShapeDtypeStruct((N, D), jnp.float32),
    )(h, parts, W1, b1.reshape(1, D), W2, b2.reshape(1, D),
      gamma.reshape(1, D), beta.reshape(1, D))


def kernel(h, edge_index, W1, b1, W2, b2, gamma, beta):
    pad = E_PAD - E
    src = jnp.concatenate([edge_index[0], jnp.zeros((pad,), jnp.int32)])
    trash = N + jnp.arange(pad, dtype=jnp.int32) % (N_ACC - N)
    dst = jnp.concatenate([edge_index[1], trash])
    nw = NC * NS

    def to_worker_chunks(flat):
        ch = flat.reshape(N_CHUNKS, K)
        c0 = ch[: NS * CH0].reshape(NS, CH0, K)
        c1 = ch[NS * CH0 :].reshape(NS, CH1, K)
        arr = jnp.zeros((nw, CHMAX, K), jnp.int32)
        arr = arr.at[:NS, :CH0].set(c0).at[NS:, :CH1].set(c1)
        return arr.reshape(nw, CHMAX, 1, K)

    src = to_worker_chunks(src)
    dst = to_worker_chunks(dst)
    zeros_init = jnp.zeros((ROWS_PER_SUB, D), jnp.float32)
    parts = _sc_segment_sum(h, src, dst, zeros_init)
    return _tc_dense(h, parts, W1, b1, W2, b2, gamma, beta)


# speedup vs baseline: 1.1245x; 1.0739x over previous
import functools

import jax
import jax.numpy as jnp
from jax import lax
from jax.experimental import pallas as pl
from jax.experimental.pallas import tpu as pltpu
from jax.experimental.pallas import tpu_sc as plsc

N = 10000
E = 320000
D = 128

NC = 2
NS = 16
K = 128

CHUNKS_PER_WORKER = 79
E_PAD = NC * NS * CHUNKS_PER_WORKER * K
ROWS_PER_SUB = 632
N_ACC = ROWS_PER_SUB * NS


def _sc_segment_sum(h, src, dst, zeros_init):
    mesh = plsc.VectorSubcoreMesh(core_axis_name="c", subcore_axis_name="s")

    @functools.partial(
        pl.kernel,
        out_type=jax.ShapeDtypeStruct((NC, N_ACC, D), jnp.float32),
        mesh=mesh,
        scratch_types=[
            pltpu.VMEM_SHARED((N_ACC, D), jnp.float32),
            pltpu.VMEM((K,), jnp.int32),
            pltpu.VMEM((K,), jnp.int32),
            pltpu.VMEM((K, D), jnp.float32),
            pltpu.SemaphoreType.DMA,
        ],
    )
    def seg_sum(h_hbm, src_hbm, dst_hbm, zero_hbm, out_hbm, acc, src_v, dst_v,
                rows_v, sem):
        c = lax.axis_index("c")
        s = lax.axis_index("s")
        wid = c * NS + s

        pltpu.sync_copy(zero_hbm, acc.at[pl.ds(s * ROWS_PER_SUB, ROWS_PER_SUB)])
        plsc.subcore_barrier()

        def body(i, carry):
            off = (wid * CHUNKS_PER_WORKER + i) * K
            pltpu.sync_copy(src_hbm.at[pl.ds(off, K)], src_v)
            pltpu.async_copy(h_hbm.at[src_v], rows_v, sem).wait()
            pltpu.sync_copy(dst_hbm.at[pl.ds(off, K)], dst_v)
            pltpu.sync_copy(rows_v, acc.at[dst_v], add=True)
            return carry

        lax.fori_loop(0, CHUNKS_PER_WORKER, body, 0)
        plsc.subcore_barrier()

        pltpu.sync_copy(acc.at[pl.ds(s * ROWS_PER_SUB, ROWS_PER_SUB)],
                        out_hbm.at[c, pl.ds(s * ROWS_PER_SUB, ROWS_PER_SUB)])

    return seg_sum(h, src, dst, zeros_init)


def _tc_dense(h, parts, W1, b1, W2, b2, gamma, beta):
    def body(h_ref, p_ref, W1_ref, b1_ref, W2_ref, b2_ref, g_ref, bt_ref,
             out_ref):
        hh = h_ref[...]
        x = hh + p_ref[0, :N, :] + p_ref[1, :N, :]
        y = jnp.maximum(
            jnp.dot(x, W1_ref[...], preferred_element_type=jnp.float32)
            + b1_ref[...], 0.0)
        z = (jnp.dot(y, W2_ref[...], preferred_element_type=jnp.float32)
             + b2_ref[...])
        mean = jnp.mean(z, axis=0, keepdims=True)
        zc = z - mean
        var = jnp.mean(zc * zc, axis=0, keepdims=True)
        zn = zc * jax.lax.rsqrt(var + 1e-5) * g_ref[...] + bt_ref[...]
        out_ref[...] = hh + jnp.maximum(zn, 0.0)

    return pl.pallas_call(
        body,
        out_shape=jax.ShapeDtypeStruct((N, D), jnp.float32),
    )(h, parts, W1, b1.reshape(1, D), W2, b2.reshape(1, D),
      gamma.reshape(1, D), beta.reshape(1, D))


def kernel(h, edge_index, W1, b1, W2, b2, gamma, beta):
    pad = E_PAD - E
    src = jnp.concatenate([edge_index[0], jnp.zeros((pad,), jnp.int32)])
    trash = N + jnp.arange(pad, dtype=jnp.int32) % (N_ACC - N)
    dst = jnp.concatenate([edge_index[1], trash])
    zeros_init = jnp.zeros((ROWS_PER_SUB, D), jnp.float32)
    parts = _sc_segment_sum(h, src, dst, zeros_init)
    return _tc_dense(h, parts, W1, b1, W2, b2, gamma, beta)
